# Initial kernel scaffold; baseline (speedup 1.0000x reference)
import functools

import jax
import jax.numpy as jnp
from jax import lax
from jax.experimental import pallas as pl
from jax.experimental.pallas import tpu as pltpu
from jax.experimental.pallas import tpu_sc as plsc

N = 10000
NNZ = 320000
NC = 2
NS = 16
NW = NC * NS
K = 80
NCH = NNZ // (NW * K)
RPS = N // NS

_mesh = plsc.VectorSubcoreMesh(core_axis_name="c", subcore_axis_name="s")


def _sc_segment_sum(table, src_idx, dst_idx, zeros_nd):
    D = table.shape[1]

    @functools.partial(
        pl.kernel,
        out_type=(jax.ShapeDtypeStruct((N, D), jnp.float32),
                  jax.ShapeDtypeStruct((N, D), jnp.float32)),
        mesh=_mesh,
        scratch_types=[
            pltpu.VMEM((NCH, K), jnp.int32),
            pltpu.VMEM((NCH, K), jnp.int32),
            pltpu.VMEM((K, D), jnp.float32),
            pltpu.VMEM((K, D), jnp.float32),
            pltpu.VMEM_SHARED((N, D), jnp.float32),
            pltpu.SemaphoreType.DMA,
            pltpu.SemaphoreType.DMA,
        ],
    )
    def run(t_hbm, src_hbm, dst_hbm, z_hbm, out0, out1,
            src_v, dst_v, rb0, rb1, acc, g0, g1):
        c = lax.axis_index("c")
        s = lax.axis_index("s")
        wid = c * NS + s
        pltpu.sync_copy(src_hbm.at[pl.ds(wid * NCH, NCH)], src_v)
        pltpu.sync_copy(dst_hbm.at[pl.ds(wid * NCH, NCH)], dst_v)
        row0 = s * RPS
        pltpu.sync_copy(z_hbm.at[pl.ds(row0, RPS)], acc.at[pl.ds(row0, RPS)])
        plsc.subcore_barrier()

        pltpu.async_copy(t_hbm.at[src_v.at[0]], rb0, g0)

        @pl.loop(0, NCH - 1, step=2)
        def _(i):
            pltpu.make_async_copy(t_hbm.at[src_v.at[i]], rb0, g0).wait()
            pltpu.async_copy(t_hbm.at[src_v.at[i + 1]], rb1, g1)
            pltpu.sync_copy(rb0, acc.at[dst_v.at[i]], add=True)
            pltpu.make_async_copy(t_hbm.at[src_v.at[i + 1]], rb1, g1).wait()
            pltpu.async_copy(t_hbm.at[src_v.at[i + 2]], rb0, g0)
            pltpu.sync_copy(rb1, acc.at[dst_v.at[i + 1]], add=True)

        pltpu.make_async_copy(t_hbm.at[src_v.at[NCH - 1]], rb0, g0).wait()
        pltpu.sync_copy(rb0, acc.at[dst_v.at[NCH - 1]], add=True)
        plsc.subcore_barrier()

        @pl.when(c == 0)
        def _():
            pltpu.sync_copy(acc.at[pl.ds(row0, RPS)], out0.at[pl.ds(row0, RPS)])

        @pl.when(c == 1)
        def _():
            pltpu.sync_copy(acc.at[pl.ds(row0, RPS)], out1.at[pl.ds(row0, RPS)])

    return run(table, src_idx, dst_idx, zeros_nd)


def _sc_degrees(eidx, vidx, ones_k, zeros_n16):
    shp = jax.ShapeDtypeStruct((N, 16), jnp.float32)

    @functools.partial(
        pl.kernel,
        out_type=(shp, shp, shp, shp),
        mesh=_mesh,
        scratch_types=[
            pltpu.VMEM((NCH, K), jnp.int32),
            pltpu.VMEM((NCH, K), jnp.int32),
            pltpu.VMEM((K, 16), jnp.float32),
            pltpu.VMEM_SHARED((N, 16), jnp.float32),
            pltpu.VMEM_SHARED((N, 16), jnp.float32),
        ],
    )
    def run(e_hbm, v_hbm, ones_hbm, z_hbm, ce0, ce1, cv0, cv1,
            e_v, v_v, ones_v, acc_e, acc_v):
        c = lax.axis_index("c")
        s = lax.axis_index("s")
        wid = c * NS + s
        pltpu.sync_copy(e_hbm.at[pl.ds(wid * NCH, NCH)], e_v)
        pltpu.sync_copy(v_hbm.at[pl.ds(wid * NCH, NCH)], v_v)
        pltpu.sync_copy(ones_hbm, ones_v)
        row0 = s * RPS
        pltpu.sync_copy(z_hbm.at[pl.ds(row0, RPS)], acc_e.at[pl.ds(row0, RPS)])
        pltpu.sync_copy(z_hbm.at[pl.ds(row0, RPS)], acc_v.at[pl.ds(row0, RPS)])
        plsc.subcore_barrier()

        @pl.loop(0, NCH)
        def _(i):
            pltpu.sync_copy(ones_v, acc_e.at[e_v.at[i]], add=True)
            pltpu.sync_copy(ones_v, acc_v.at[v_v.at[i]], add=True)

        plsc.subcore_barrier()

        @pl.when(c == 0)
        def _():
            pltpu.sync_copy(acc_e.at[pl.ds(row0, RPS)], ce0.at[pl.ds(row0, RPS)])
            pltpu.sync_copy(acc_v.at[pl.ds(row0, RPS)], cv0.at[pl.ds(row0, RPS)])

        @pl.when(c == 1)
        def _():
            pltpu.sync_copy(acc_e.at[pl.ds(row0, RPS)], ce1.at[pl.ds(row0, RPS)])
            pltpu.sync_copy(acc_v.at[pl.ds(row0, RPS)], cv1.at[pl.ds(row0, RPS)])

    return run(eidx, vidx, ones_k, zeros_n16)


_BLK = 1000


def _tc_matmul(X, W, b):
    n, d_in = X.shape
    d_out = W.shape[1]

    def body(x_ref, w_ref, b_ref, o_ref):
        o_ref[...] = lax.dot_general(
            x_ref[...], w_ref[...], (((1,), (0,)), ((), ())),
            preferred_element_type=jnp.float32,
            precision=lax.Precision.HIGHEST) + b_ref[...]

    return pl.pallas_call(
        body,
        grid=(n // _BLK,),
        in_specs=[pl.BlockSpec((_BLK, d_in), lambda i: (i, 0)),
                  pl.BlockSpec((d_in, d_out), lambda i: (0, 0)),
                  pl.BlockSpec((1, d_out), lambda i: (0, 0))],
        out_specs=pl.BlockSpec((_BLK, d_out), lambda i: (i, 0)),
        out_shape=jax.ShapeDtypeStruct((n, d_out), jnp.float32),
    )(X, W, b.reshape(1, -1))


def _tc_combine(p0, p1, c0, c1, relu):
    n, d = p0.shape

    def body(p0_ref, p1_ref, c0_ref, c1_ref, o_ref):
        cnt = jnp.maximum(c0_ref[...][:, 0:1] + c1_ref[...][:, 0:1], 1.0)
        r = (p0_ref[...] + p1_ref[...]) / cnt
        if relu:
            r = jnp.maximum(r, 0.0)
        o_ref[...] = r

    return pl.pallas_call(
        body,
        grid=(n // _BLK,),
        in_specs=[pl.BlockSpec((_BLK, d), lambda i: (i, 0)),
                  pl.BlockSpec((_BLK, d), lambda i: (i, 0)),
                  pl.BlockSpec((_BLK, 16), lambda i: (i, 0)),
                  pl.BlockSpec((_BLK, 16), lambda i: (i, 0))],
        out_specs=pl.BlockSpec((_BLK, d), lambda i: (i, 0)),
        out_shape=jax.ShapeDtypeStruct((n, d), jnp.float32),
    )(p0, p1, c0, c1)


def _tc_boundary(q0, q1, c0, c1, x_res, W, b):
    n, d = q0.shape
    d_out = W.shape[1]
    with_res = x_res is not None

    def body(*refs):
        if with_res:
            q0_ref, q1_ref, c0_ref, c1_ref, xr_ref, w_ref, b_ref, z_ref, y_ref = refs
        else:
            q0_ref, q1_ref, c0_ref, c1_ref, w_ref, b_ref, z_ref, y_ref = refs
        cnt = jnp.maximum(c0_ref[...][:, 0:1] + c1_ref[...][:, 0:1], 1.0)
        z = jnp.maximum((q0_ref[...] + q1_ref[...]) / cnt, 0.0)
        if with_res:
            z = z + xr_ref[...]
        z_ref[...] = z
        y_ref[...] = lax.dot_general(
            z, w_ref[...], (((1,), (0,)), ((), ())),
            preferred_element_type=jnp.float32,
            precision=lax.Precision.HIGHEST) + b_ref[...]

    in_specs = [pl.BlockSpec((_BLK, d), lambda i: (i, 0)),
                pl.BlockSpec((_BLK, d), lambda i: (i, 0)),
                pl.BlockSpec((_BLK, 16), lambda i: (i, 0)),
                pl.BlockSpec((_BLK, 16), lambda i: (i, 0))]
    args = [q0, q1, c0, c1]
    if with_res:
        in_specs.append(pl.BlockSpec((_BLK, d), lambda i: (i, 0)))
        args.append(x_res)
    in_specs += [pl.BlockSpec((d, d_out), lambda i: (0, 0)),
                 pl.BlockSpec((1, d_out), lambda i: (0, 0))]
    args += [W, b.reshape(1, -1)]

    return pl.pallas_call(
        body,
        grid=(n // _BLK,),
        in_specs=in_specs,
        out_specs=[pl.BlockSpec((_BLK, d), lambda i: (i, 0)),
                   pl.BlockSpec((_BLK, d_out), lambda i: (i, 0))],
        out_shape=[jax.ShapeDtypeStruct((n, d), jnp.float32),
                   jax.ShapeDtypeStruct((n, d_out), jnp.float32)],
    )(*args)


def kernel(X, edge_index, W1, b1, W2, b2, W3, b3):
    vids = edge_index[0].reshape(NW * NCH, K)
    eids = edge_index[1].reshape(NW * NCH, K)
    zeros128 = jnp.zeros((N, 128), jnp.float32)
    zeros64 = jnp.zeros((N, 64), jnp.float32)
    zeros16 = jnp.zeros((N, 16), jnp.float32)
    ones_k = jnp.ones((K, 16), jnp.float32)

    ce0, ce1, cv0, cv1 = _sc_degrees(eids, vids, ones_k, zeros16)

    y1 = _tc_matmul(X, W1, b1)
    p0, p1 = _sc_segment_sum(y1, vids, eids, zeros128)
    e1 = _tc_combine(p0, p1, ce0, ce1, relu=False)
    q0, q1 = _sc_segment_sum(e1, eids, vids, zeros128)
    x1, y2 = _tc_boundary(q0, q1, cv0, cv1, None, W2, b2)

    p0, p1 = _sc_segment_sum(y2, vids, eids, zeros128)
    e2 = _tc_combine(p0, p1, ce0, ce1, relu=False)
    q0, q1 = _sc_segment_sum(e2, eids, vids, zeros128)
    _, y3 = _tc_boundary(q0, q1, cv0, cv1, x1, W3, b3)

    p0, p1 = _sc_segment_sum(y3, vids, eids, zeros64)
    e3 = _tc_combine(p0, p1, ce0, ce1, relu=False)
    q0, q1 = _sc_segment_sum(e3, eids, vids, zeros64)
    x3 = _tc_combine(q0, q1, cv0, cv1, relu=True)
    return x3


# speedup vs baseline: 4.4520x; 4.4520x over previous
import functools

import jax
import jax.numpy as jnp
from jax import lax
from jax.experimental import pallas as pl
from jax.experimental.pallas import tpu as pltpu
from jax.experimental.pallas import tpu_sc as plsc

N = 10000
NNZ = 320000
NC = 2
NS = 16
K = 80
NCH = NNZ // (NS * K)
NACC = N // NC
NTR = 128
NPAD = NACC + NTR
ZRA, ZRB = 320, NPAD - 15 * 320
ORA, ORB = 312, NACC - 15 * 312

_mesh = plsc.VectorSubcoreMesh(core_axis_name="c", subcore_axis_name="s")


def _sc_segment_sum(table, src_idx, dst_idx, zeros_pad):
    D = table.shape[1]

    @functools.partial(
        pl.kernel,
        out_type=jax.ShapeDtypeStruct((N, D), jnp.float32),
        mesh=_mesh,
        scratch_types=[
            pltpu.VMEM((NCH, K), jnp.int32),
            pltpu.VMEM((NCH, K), jnp.int32),
            pltpu.VMEM((K, D), jnp.float32),
            pltpu.VMEM((K, D), jnp.float32),
            pltpu.VMEM_SHARED((NPAD, D), jnp.float32),
            pltpu.SemaphoreType.DMA,
            pltpu.SemaphoreType.DMA,
        ],
    )
    def run(t_hbm, src_hbm, dst_hbm, z_hbm, out,
            src_v, dst_v, rb0, rb1, acc, g0, g1):
        c = lax.axis_index("c")
        s = lax.axis_index("s")
        pltpu.sync_copy(src_hbm.at[s], src_v)
        pltpu.sync_copy(dst_hbm.at[s], dst_v)

        @pl.when(s < 15)
        def _():
            pltpu.sync_copy(z_hbm.at[pl.ds(s * ZRA, ZRA)],
                            acc.at[pl.ds(s * ZRA, ZRA)])

        @pl.when(s == 15)
        def _():
            pltpu.sync_copy(z_hbm.at[pl.ds(15 * ZRA, ZRB)],
                            acc.at[pl.ds(15 * ZRA, ZRB)])

        base = c * NACC

        @pl.loop(0, NCH)
        def _(j):
            tr = NACC + 16 * lax.rem(j, 8) + lax.iota(jnp.int32, 16)
            for l in range(K // 16):
                d = dst_v[j, pl.ds(l * 16, 16)]
                local = d - base
                inr = (local >= 0) & (local < NACC)
                dst_v[j, pl.ds(l * 16, 16)] = jnp.where(inr, local, tr)

        plsc.subcore_barrier()

        pltpu.async_copy(t_hbm.at[src_v.at[0]], rb0, g0)

        @pl.loop(0, NCH - 2, step=2)
        def _(i):
            pltpu.make_async_copy(t_hbm.at[src_v.at[i]], rb0, g0).wait()
            pltpu.async_copy(t_hbm.at[src_v.at[i + 1]], rb1, g1)
            pltpu.sync_copy(rb0, acc.at[dst_v.at[i]], add=True)
            pltpu.make_async_copy(t_hbm.at[src_v.at[i + 1]], rb1, g1).wait()
            pltpu.async_copy(t_hbm.at[src_v.at[i + 2]], rb0, g0)
            pltpu.sync_copy(rb1, acc.at[dst_v.at[i + 1]], add=True)

        pltpu.make_async_copy(t_hbm.at[src_v.at[NCH - 2]], rb0, g0).wait()
        pltpu.async_copy(t_hbm.at[src_v.at[NCH - 1]], rb1, g1)
        pltpu.sync_copy(rb0, acc.at[dst_v.at[NCH - 2]], add=True)
        pltpu.make_async_copy(t_hbm.at[src_v.at[NCH - 1]], rb1, g1).wait()
        pltpu.sync_copy(rb1, acc.at[dst_v.at[NCH - 1]], add=True)
        plsc.subcore_barrier()

        @pl.when(s < 15)
        def _():
            pltpu.sync_copy(acc.at[pl.ds(s * ORA, ORA)],
                            out.at[pl.ds(base + s * ORA, ORA)])

        @pl.when(s == 15)
        def _():
            pltpu.sync_copy(acc.at[pl.ds(15 * ORA, ORB)],
                            out.at[pl.ds(base + 15 * ORA, ORB)])

    return run(table, src_idx, dst_idx, zeros_pad)


def _sc_degrees(eidx, vidx, ones_k, zeros_pad):
    shp = jax.ShapeDtypeStruct((N, 128), jnp.float32)

    @functools.partial(
        pl.kernel,
        out_type=(shp, shp),
        mesh=_mesh,
        scratch_types=[
            pltpu.VMEM((NCH, K), jnp.int32),
            pltpu.VMEM((K, 128), jnp.float32),
            pltpu.VMEM_SHARED((NPAD, 128), jnp.float32),
        ],
    )
    def run(e_hbm, v_hbm, ones_hbm, z_hbm, cnt_e, cnt_v,
            idx_v, ones_v, acc):
        c = lax.axis_index("c")
        s = lax.axis_index("s")
        base = c * NACC
        pltpu.sync_copy(ones_hbm, ones_v)
        for idx_hbm, out in ((e_hbm, cnt_e), (v_hbm, cnt_v)):
            pltpu.sync_copy(idx_hbm.at[s], idx_v)

            @pl.when(s < 15)
            def _():
                pltpu.sync_copy(z_hbm.at[pl.ds(s * ZRA, ZRA)],
                                acc.at[pl.ds(s * ZRA, ZRA)])

            @pl.when(s == 15)
            def _():
                pltpu.sync_copy(z_hbm.at[pl.ds(15 * ZRA, ZRB)],
                                acc.at[pl.ds(15 * ZRA, ZRB)])

            @pl.loop(0, NCH)
            def _(j):
                tr = NACC + 16 * lax.rem(j, 8) + lax.iota(jnp.int32, 16)
                for l in range(K // 16):
                    d = idx_v[j, pl.ds(l * 16, 16)]
                    local = d - base
                    inr = (local >= 0) & (local < NACC)
                    idx_v[j, pl.ds(l * 16, 16)] = jnp.where(inr, local, tr)

            plsc.subcore_barrier()

            @pl.loop(0, NCH)
            def _(j):
                pltpu.sync_copy(ones_v, acc.at[idx_v.at[j]], add=True)

            plsc.subcore_barrier()

            @pl.when(s < 15)
            def _():
                pltpu.sync_copy(acc.at[pl.ds(s * ORA, ORA)],
                                out.at[pl.ds(base + s * ORA, ORA)])

            @pl.when(s == 15)
            def _():
                pltpu.sync_copy(acc.at[pl.ds(15 * ORA, ORB)],
                                out.at[pl.ds(base + 15 * ORA, ORB)])

            plsc.subcore_barrier()

    return run(eidx, vidx, ones_k, zeros_pad)


_BLK = 1000


def _rowspec(d):
    return pl.BlockSpec((_BLK, d), lambda i: (i, 0))


def _dot(a, b):
    return lax.dot_general(a, b, (((1,), (0,)), ((), ())),
                           preferred_element_type=jnp.float32,
                           precision=lax.Precision.HIGHEST)


def _tc_matmul(X, W, b, d_pad):
    n, d_in = X.shape
    d_out = W.shape[1]

    def body(x_ref, w_ref, b_ref, o_ref):
        y = _dot(x_ref[...], w_ref[...]) + b_ref[...]
        if d_pad > d_out:
            y = jnp.concatenate(
                [y, jnp.zeros((_BLK, d_pad - d_out), jnp.float32)], axis=1)
        o_ref[...] = y

    return pl.pallas_call(
        body,
        grid=(n // _BLK,),
        in_specs=[_rowspec(d_in),
                  pl.BlockSpec((d_in, d_out), lambda i: (0, 0)),
                  pl.BlockSpec((1, d_out), lambda i: (0, 0))],
        out_specs=_rowspec(d_pad),
        out_shape=jax.ShapeDtypeStruct((n, d_pad), jnp.float32),
    )(X, W, b.reshape(1, -1))


def _tc_scale(ssum, cnt, relu, d_out=None):
    n, d = ssum.shape
    d_out = d_out or d

    def body(s_ref, c_ref, o_ref):
        cnt_col = jnp.maximum(c_ref[...][:, 0:1], 1.0)
        r = s_ref[...][:, :d_out] / cnt_col
        if relu:
            r = jnp.maximum(r, 0.0)
        o_ref[...] = r

    return pl.pallas_call(
        body,
        grid=(n // _BLK,),
        in_specs=[_rowspec(d), _rowspec(128)],
        out_specs=_rowspec(d_out),
        out_shape=jax.ShapeDtypeStruct((n, d_out), jnp.float32),
    )(ssum, cnt)


def _tc_boundary(ssum, cnt, x_res, W, b, d_pad):
    n, d = ssum.shape
    d_out = W.shape[1]
    with_res = x_res is not None

    def body(*refs):
        if with_res:
            s_ref, c_ref, xr_ref, w_ref, b_ref, z_ref, y_ref = refs
        else:
            s_ref, c_ref, w_ref, b_ref, z_ref, y_ref = refs
        cnt_col = jnp.maximum(c_ref[...][:, 0:1], 1.0)
        z = jnp.maximum(s_ref[...] / cnt_col, 0.0)
        if with_res:
            z = z + xr_ref[...]
        z_ref[...] = z
        y = _dot(z, w_ref[...]) + b_ref[...]
        if d_pad > d_out:
            y = jnp.concatenate(
                [y, jnp.zeros((_BLK, d_pad - d_out), jnp.float32)], axis=1)
        y_ref[...] = y

    in_specs = [_rowspec(d), _rowspec(128)]
    args = [ssum, cnt]
    if with_res:
        in_specs.append(_rowspec(d))
        args.append(x_res)
    in_specs += [pl.BlockSpec((d, d_out), lambda i: (0, 0)),
                 pl.BlockSpec((1, d_out), lambda i: (0, 0))]
    args += [W, b.reshape(1, -1)]

    return pl.pallas_call(
        body,
        grid=(n // _BLK,),
        in_specs=in_specs,
        out_specs=[_rowspec(d), _rowspec(d_pad)],
        out_shape=[jax.ShapeDtypeStruct((n, d), jnp.float32),
                   jax.ShapeDtypeStruct((n, d_pad), jnp.float32)],
    )(*args)


def kernel(X, edge_index, W1, b1, W2, b2, W3, b3):
    vids = edge_index[0].reshape(NS, NCH, K)
    eids = edge_index[1].reshape(NS, NCH, K)
    zeros_pad = jnp.zeros((NPAD, 128), jnp.float32)
    ones_k = jnp.ones((K, 128), jnp.float32)

    cnt_e, cnt_v = _sc_degrees(eids, vids, ones_k, zeros_pad)

    y1 = _tc_matmul(X, W1, b1, 128)
    s = _sc_segment_sum(y1, vids, eids, zeros_pad)
    e1 = _tc_scale(s, cnt_e, relu=False)
    s = _sc_segment_sum(e1, eids, vids, zeros_pad)
    x1, y2 = _tc_boundary(s, cnt_v, None, W2, b2, 128)

    s = _sc_segment_sum(y2, vids, eids, zeros_pad)
    e2 = _tc_scale(s, cnt_e, relu=False)
    s = _sc_segment_sum(e2, eids, vids, zeros_pad)
    _, y3 = _tc_boundary(s, cnt_v, x1, W3, b3, 128)

    s = _sc_segment_sum(y3, vids, eids, zeros_pad)
    e3 = _tc_scale(s, cnt_e, relu=False)
    s = _sc_segment_sum(e3, eids, vids, zeros_pad)
    x3 = _tc_scale(s, cnt_v, relu=True, d_out=64)
    return x3


# speedup vs baseline: 7.9870x; 1.7940x over previous
import functools

import jax
import jax.numpy as jnp
from jax import lax
from jax.experimental import pallas as pl
from jax.experimental.pallas import tpu as pltpu
from jax.experimental.pallas import tpu_sc as plsc

N = 10000
NNZ = 320000
NC = 2
NS = 16
K = 100
NCH = NNZ // (NC * NS * K)
NHV = 2
NH = NCH // NHV
RA, RB = 624, N - 15 * 624

_mesh = plsc.VectorSubcoreMesh(core_axis_name="c", subcore_axis_name="s")


def _sc_segment_sum(table, src_idx, dst_idx, zeros_pad):
    D = table.shape[1]

    @functools.partial(
        pl.kernel,
        out_type=jax.ShapeDtypeStruct((NC * N, D), jnp.float32),
        mesh=_mesh,
        scratch_types=[
            pltpu.VMEM_SHARED((N, D), jnp.float32),
            pltpu.VMEM((NH, K), jnp.int32),
            pltpu.VMEM((NH, K), jnp.int32),
            pltpu.VMEM((K, D), jnp.float32),
            pltpu.VMEM((K, D), jnp.float32),
            pltpu.SemaphoreType.DMA,
            pltpu.SemaphoreType.DMA,
        ],
    )
    def run(t_hbm, src_hbm, dst_hbm, z_hbm, out,
            acc, src_v, dst_v, rb0, rb1, g0, g1):
        c = lax.axis_index("c")
        s = lax.axis_index("s")

        @pl.when(s < 15)
        def _():
            pltpu.sync_copy(z_hbm.at[pl.ds(s * RA, RA)],
                            acc.at[pl.ds(s * RA, RA)])

        @pl.when(s == 15)
        def _():
            pltpu.sync_copy(z_hbm.at[pl.ds(15 * RA, RB)],
                            acc.at[pl.ds(15 * RA, RB)])

        plsc.subcore_barrier()

        for h in range(NHV):
            pltpu.sync_copy(src_hbm.at[(c * NS + s) * NHV + h], src_v)
            pltpu.sync_copy(dst_hbm.at[(c * NS + s) * NHV + h], dst_v)

            pltpu.async_copy(t_hbm.at[src_v.at[0]], rb0, g0)

            @pl.loop(0, NH - 2, step=2)
            def _(i):
                pltpu.make_async_copy(t_hbm.at[src_v.at[i]], rb0, g0).wait()
                pltpu.async_copy(t_hbm.at[src_v.at[i + 1]], rb1, g1)
                pltpu.sync_copy(rb0, acc.at[dst_v.at[i]], add=True)
                pltpu.make_async_copy(t_hbm.at[src_v.at[i + 1]], rb1, g1).wait()
                pltpu.async_copy(t_hbm.at[src_v.at[i + 2]], rb0, g0)
                pltpu.sync_copy(rb1, acc.at[dst_v.at[i + 1]], add=True)

            pltpu.make_async_copy(t_hbm.at[src_v.at[NH - 2]], rb0, g0).wait()
            pltpu.async_copy(t_hbm.at[src_v.at[NH - 1]], rb1, g1)
            pltpu.sync_copy(rb0, acc.at[dst_v.at[NH - 2]], add=True)
            pltpu.make_async_copy(t_hbm.at[src_v.at[NH - 1]], rb1, g1).wait()
            pltpu.sync_copy(rb1, acc.at[dst_v.at[NH - 1]], add=True)

        plsc.subcore_barrier()

        @pl.when(s < 15)
        def _():
            pltpu.sync_copy(acc.at[pl.ds(s * RA, RA)],
                            out.at[pl.ds(c * N + s * RA, RA)])

        @pl.when(s == 15)
        def _():
            pltpu.sync_copy(acc.at[pl.ds(15 * RA, RB)],
                            out.at[pl.ds(c * N + 15 * RA, RB)])

    return run(table, src_idx, dst_idx, zeros_pad)


def _sc_degrees(eidx, vidx, ones_k, zeros_pad):
    shp = jax.ShapeDtypeStruct((NC * N, 128), jnp.float32)

    @functools.partial(
        pl.kernel,
        out_type=(shp, shp),
        mesh=_mesh,
        scratch_types=[
            pltpu.VMEM_SHARED((N, 128), jnp.float32),
            pltpu.VMEM((NCH, K), jnp.int32),
            pltpu.VMEM((K, 128), jnp.float32),
        ],
    )
    def run(e_hbm, v_hbm, ones_hbm, z_hbm, cnt_e, cnt_v,
            acc, idx_v, ones_v):
        c = lax.axis_index("c")
        s = lax.axis_index("s")
        pltpu.sync_copy(ones_hbm, ones_v)
        for idx_hbm, out in ((e_hbm, cnt_e), (v_hbm, cnt_v)):
            pltpu.sync_copy(idx_hbm.at[c * NS + s], idx_v)

            @pl.when(s < 15)
            def _():
                pltpu.sync_copy(z_hbm.at[pl.ds(s * RA, RA)],
                                acc.at[pl.ds(s * RA, RA)])

            @pl.when(s == 15)
            def _():
                pltpu.sync_copy(z_hbm.at[pl.ds(15 * RA, RB)],
                                acc.at[pl.ds(15 * RA, RB)])

            plsc.subcore_barrier()

            @pl.loop(0, NCH)
            def _(j):
                pltpu.sync_copy(ones_v, acc.at[idx_v.at[j]], add=True)

            plsc.subcore_barrier()

            @pl.when(s < 15)
            def _():
                pltpu.sync_copy(acc.at[pl.ds(s * RA, RA)],
                                out.at[pl.ds(c * N + s * RA, RA)])

            @pl.when(s == 15)
            def _():
                pltpu.sync_copy(acc.at[pl.ds(15 * RA, RB)],
                                out.at[pl.ds(c * N + 15 * RA, RB)])

            plsc.subcore_barrier()

    return run(eidx, vidx, ones_k, zeros_pad)


_BLK = 1000


def _rowspec(d):
    return pl.BlockSpec((_BLK, d), lambda i: (i, 0))


def _dot(a, b):
    return lax.dot_general(a, b, (((1,), (0,)), ((), ())),
                           preferred_element_type=jnp.float32,
                           precision=lax.Precision.HIGHEST)


def _tc_matmul(X, W, b, d_pad):
    n, d_in = X.shape
    d_out = W.shape[1]

    def body(x_ref, w_ref, b_ref, o_ref):
        y = _dot(x_ref[...], w_ref[...]) + b_ref[...]
        if d_pad > d_out:
            y = jnp.concatenate(
                [y, jnp.zeros((_BLK, d_pad - d_out), jnp.float32)], axis=1)
        o_ref[...] = y

    return pl.pallas_call(
        body,
        grid=(n // _BLK,),
        in_specs=[_rowspec(d_in),
                  pl.BlockSpec((d_in, d_out), lambda i: (0, 0)),
                  pl.BlockSpec((1, d_out), lambda i: (0, 0))],
        out_specs=_rowspec(d_pad),
        out_shape=jax.ShapeDtypeStruct((n, d_pad), jnp.float32),
    )(X, W, b.reshape(1, -1))


def _tc_scale(ssum, cnt, relu, d_out=None):
    d = ssum.shape[1]
    d_out = d_out or d

    def body(s0_ref, s1_ref, c0_ref, c1_ref, o_ref):
        cnt_col = jnp.maximum(c0_ref[...][:, 0:1] + c1_ref[...][:, 0:1], 1.0)
        r = (s0_ref[...][:, :d_out] + s1_ref[...][:, :d_out]) / cnt_col
        if relu:
            r = jnp.maximum(r, 0.0)
        o_ref[...] = r

    return pl.pallas_call(
        body,
        grid=(N // _BLK,),
        in_specs=[_rowspec(d), _rowspec(d), _rowspec(128), _rowspec(128)],
        out_specs=_rowspec(d_out),
        out_shape=jax.ShapeDtypeStruct((N, d_out), jnp.float32),
    )(ssum[:N], ssum[N:], cnt[:N], cnt[N:])


def _tc_boundary(ssum, cnt, x_res, W, b, d_pad):
    d = ssum.shape[1]
    d_out = W.shape[1]
    with_res = x_res is not None

    def body(*refs):
        if with_res:
            s0_ref, s1_ref, c0_ref, c1_ref, xr_ref, w_ref, b_ref, z_ref, y_ref = refs
        else:
            s0_ref, s1_ref, c0_ref, c1_ref, w_ref, b_ref, z_ref, y_ref = refs
        cnt_col = jnp.maximum(c0_ref[...][:, 0:1] + c1_ref[...][:, 0:1], 1.0)
        z = jnp.maximum((s0_ref[...] + s1_ref[...]) / cnt_col, 0.0)
        if with_res:
            z = z + xr_ref[...]
        z_ref[...] = z
        y = _dot(z, w_ref[...]) + b_ref[...]
        if d_pad > d_out:
            y = jnp.concatenate(
                [y, jnp.zeros((_BLK, d_pad - d_out), jnp.float32)], axis=1)
        y_ref[...] = y

    in_specs = [_rowspec(d), _rowspec(d), _rowspec(128), _rowspec(128)]
    args = [ssum[:N], ssum[N:], cnt[:N], cnt[N:]]
    if with_res:
        in_specs.append(_rowspec(d))
        args.append(x_res)
    in_specs += [pl.BlockSpec((d, d_out), lambda i: (0, 0)),
                 pl.BlockSpec((1, d_out), lambda i: (0, 0))]
    args += [W, b.reshape(1, -1)]

    return pl.pallas_call(
        body,
        grid=(N // _BLK,),
        in_specs=in_specs,
        out_specs=[_rowspec(d), _rowspec(d_pad)],
        out_shape=[jax.ShapeDtypeStruct((N, d), jnp.float32),
                   jax.ShapeDtypeStruct((N, d_pad), jnp.float32)],
    )(*args)


def kernel(X, edge_index, W1, b1, W2, b2, W3, b3):
    vids = edge_index[0].reshape(NC * NS * NHV, NH, K)
    eids = edge_index[1].reshape(NC * NS * NHV, NH, K)
    vids_d = edge_index[0].reshape(NC * NS, NCH, K)
    eids_d = edge_index[1].reshape(NC * NS, NCH, K)
    zeros_pad = jnp.zeros((N, 128), jnp.float32)
    zeros_cnt = jnp.zeros((N, 128), jnp.float32)
    ones_k = jnp.ones((K, 128), jnp.float32)

    cnt_e, cnt_v = _sc_degrees(eids_d, vids_d, ones_k, zeros_cnt)

    y1 = _tc_matmul(X, W1, b1, 128)
    s = _sc_segment_sum(y1, vids, eids, zeros_pad)
    e1 = _tc_scale(s, cnt_e, relu=False)
    s = _sc_segment_sum(e1, eids, vids, zeros_pad)
    x1, y2 = _tc_boundary(s, cnt_v, None, W2, b2, 128)

    s = _sc_segment_sum(y2, vids, eids, zeros_pad)
    e2 = _tc_scale(s, cnt_e, relu=False)
    s = _sc_segment_sum(e2, eids, vids, zeros_pad)
    _, y3 = _tc_boundary(s, cnt_v, x1, W3, b3, 128)

    s = _sc_segment_sum(y3, vids, eids, zeros_pad)
    e3 = _tc_scale(s, cnt_e, relu=False)
    s = _sc_segment_sum(e3, eids, vids, zeros_pad)
    x3 = _tc_scale(s, cnt_v, relu=True, d_out=64)
    return x3


# speedup vs baseline: 8.0022x; 1.0019x over previous
import functools

import jax
import jax.numpy as jnp
from jax import lax
from jax.experimental import pallas as pl
from jax.experimental.pallas import tpu as pltpu
from jax.experimental.pallas import tpu_sc as plsc

N = 10000
NNZ = 320000
NC = 2
NS = 16
K = 100
NCH = NNZ // (NC * NS * K)
NHV = 2
NH = NCH // NHV
RA, RB = 624, N - 15 * 624

_mesh = plsc.VectorSubcoreMesh(core_axis_name="c", subcore_axis_name="s")


def _sc_segment_sum(table, src_idx, dst_idx, zeros_pad):
    D = table.shape[1]

    @functools.partial(
        pl.kernel,
        out_type=jax.ShapeDtypeStruct((NC * N, D), jnp.float32),
        mesh=_mesh,
        scratch_types=[
            pltpu.VMEM_SHARED((N, D), jnp.float32),
            pltpu.VMEM((NH, K), jnp.int32),
            pltpu.VMEM((NH, K), jnp.int32),
            pltpu.VMEM((K, D), jnp.float32),
            pltpu.VMEM((K, D), jnp.float32),
            pltpu.SemaphoreType.DMA,
            pltpu.SemaphoreType.DMA,
        ],
    )
    def run(t_hbm, src_hbm, dst_hbm, z_hbm, out,
            acc, src_v, dst_v, rb0, rb1, g0, g1):
        c = lax.axis_index("c")
        s = lax.axis_index("s")

        @pl.when(s < 15)
        def _():
            pltpu.sync_copy(z_hbm.at[pl.ds(s * RA, RA)],
                            acc.at[pl.ds(s * RA, RA)])

        @pl.when(s == 15)
        def _():
            pltpu.sync_copy(z_hbm.at[pl.ds(15 * RA, RB)],
                            acc.at[pl.ds(15 * RA, RB)])

        plsc.subcore_barrier()

        for h in range(NHV):
            pltpu.sync_copy(src_hbm.at[(c * NS + s) * NHV + h], src_v)
            pltpu.sync_copy(dst_hbm.at[(c * NS + s) * NHV + h], dst_v)

            pltpu.async_copy(t_hbm.at[src_v.at[0]], rb0, g0)

            @pl.loop(0, NH - 2, step=2)
            def _(i):
                pltpu.make_async_copy(t_hbm.at[src_v.at[i]], rb0, g0).wait()
                pltpu.async_copy(t_hbm.at[src_v.at[i + 1]], rb1, g1)
                pltpu.sync_copy(rb0, acc.at[dst_v.at[i]], add=True)
                pltpu.make_async_copy(t_hbm.at[src_v.at[i + 1]], rb1, g1).wait()
                pltpu.async_copy(t_hbm.at[src_v.at[i + 2]], rb0, g0)
                pltpu.sync_copy(rb1, acc.at[dst_v.at[i + 1]], add=True)

            pltpu.make_async_copy(t_hbm.at[src_v.at[NH - 2]], rb0, g0).wait()
            pltpu.async_copy(t_hbm.at[src_v.at[NH - 1]], rb1, g1)
            pltpu.sync_copy(rb0, acc.at[dst_v.at[NH - 2]], add=True)
            pltpu.make_async_copy(t_hbm.at[src_v.at[NH - 1]], rb1, g1).wait()
            pltpu.sync_copy(rb1, acc.at[dst_v.at[NH - 1]], add=True)

        plsc.subcore_barrier()

        @pl.when(s < 15)
        def _():
            pltpu.sync_copy(acc.at[pl.ds(s * RA, RA)],
                            out.at[pl.ds(c * N + s * RA, RA)])

        @pl.when(s == 15)
        def _():
            pltpu.sync_copy(acc.at[pl.ds(15 * RA, RB)],
                            out.at[pl.ds(c * N + 15 * RA, RB)])

    return run(table, src_idx, dst_idx, zeros_pad)


def _sc_degrees(eidx, vidx, ones_k, zeros_pad):
    shp = jax.ShapeDtypeStruct((NC * N, 128), jnp.float32)

    @functools.partial(
        pl.kernel,
        out_type=(shp, shp),
        mesh=_mesh,
        scratch_types=[
            pltpu.VMEM_SHARED((N, 128), jnp.float32),
            pltpu.VMEM((NCH, K), jnp.int32),
            pltpu.VMEM((K, 128), jnp.float32),
        ],
    )
    def run(e_hbm, v_hbm, ones_hbm, z_hbm, cnt_e, cnt_v,
            acc, idx_v, ones_v):
        c = lax.axis_index("c")
        s = lax.axis_index("s")
        pltpu.sync_copy(ones_hbm, ones_v)
        for idx_hbm, out in ((e_hbm, cnt_e), (v_hbm, cnt_v)):
            pltpu.sync_copy(idx_hbm.at[c * NS + s], idx_v)

            @pl.when(s < 15)
            def _():
                pltpu.sync_copy(z_hbm.at[pl.ds(s * RA, RA)],
                                acc.at[pl.ds(s * RA, RA)])

            @pl.when(s == 15)
            def _():
                pltpu.sync_copy(z_hbm.at[pl.ds(15 * RA, RB)],
                                acc.at[pl.ds(15 * RA, RB)])

            plsc.subcore_barrier()

            @pl.loop(0, NCH)
            def _(j):
                pltpu.sync_copy(ones_v, acc.at[idx_v.at[j]], add=True)

            plsc.subcore_barrier()

            @pl.when(s < 15)
            def _():
                pltpu.sync_copy(acc.at[pl.ds(s * RA, RA)],
                                out.at[pl.ds(c * N + s * RA, RA)])

            @pl.when(s == 15)
            def _():
                pltpu.sync_copy(acc.at[pl.ds(15 * RA, RB)],
                                out.at[pl.ds(c * N + 15 * RA, RB)])

            plsc.subcore_barrier()

    return run(eidx, vidx, ones_k, zeros_pad)


_BLK = 1000


def _rowspec(d):
    return pl.BlockSpec((_BLK, d), lambda i: (i, 0))


def _dot(a, b):
    return lax.dot_general(a, b, (((1,), (0,)), ((), ())),
                           preferred_element_type=jnp.float32,
                           precision=lax.Precision.HIGHEST)


def _tc_matmul(X, W, b, d_pad):
    n, d_in = X.shape
    d_out = W.shape[1]

    def body(x_ref, w_ref, b_ref, o_ref):
        y = _dot(x_ref[...], w_ref[...]) + b_ref[...]
        if d_pad > d_out:
            y = jnp.concatenate(
                [y, jnp.zeros((_BLK, d_pad - d_out), jnp.float32)], axis=1)
        o_ref[...] = y

    return pl.pallas_call(
        body,
        grid=(n // _BLK,),
        in_specs=[_rowspec(d_in),
                  pl.BlockSpec((d_in, d_out), lambda i: (0, 0)),
                  pl.BlockSpec((1, d_out), lambda i: (0, 0))],
        out_specs=_rowspec(d_pad),
        out_shape=jax.ShapeDtypeStruct((n, d_pad), jnp.float32),
    )(X, W, b.reshape(1, -1))


def _tc_inv(cnt):
    def body(c0_ref, c1_ref, o_ref):
        t = jnp.maximum(c0_ref[...][:, 0:8] + c1_ref[...][:, 0:8], 1.0)
        o_ref[...] = 1.0 / t

    return pl.pallas_call(
        body,
        grid=(N // _BLK,),
        in_specs=[_rowspec(128), _rowspec(128)],
        out_specs=_rowspec(8),
        out_shape=jax.ShapeDtypeStruct((N, 8), jnp.float32),
    )(cnt[:N], cnt[N:])


def _tc_scale(ssum, inv, relu, d_out=None):
    d = ssum.shape[1]
    d_out = d_out or d

    def body(s0_ref, s1_ref, i_ref, o_ref):
        r = (s0_ref[...][:, :d_out] + s1_ref[...][:, :d_out]) * i_ref[...][:, 0:1]
        if relu:
            r = jnp.maximum(r, 0.0)
        o_ref[...] = r

    return pl.pallas_call(
        body,
        grid=(N // _BLK,),
        in_specs=[_rowspec(d), _rowspec(d), _rowspec(8)],
        out_specs=_rowspec(d_out),
        out_shape=jax.ShapeDtypeStruct((N, d_out), jnp.float32),
    )(ssum[:N], ssum[N:], inv)


def _tc_boundary(ssum, inv, x_res, W, b, d_pad):
    d = ssum.shape[1]
    d_out = W.shape[1]
    with_res = x_res is not None

    def body(*refs):
        if with_res:
            s0_ref, s1_ref, i_ref, xr_ref, w_ref, b_ref, z_ref, y_ref = refs
        else:
            s0_ref, s1_ref, i_ref, w_ref, b_ref, z_ref, y_ref = refs
        z = jnp.maximum((s0_ref[...] + s1_ref[...]) * i_ref[...][:, 0:1], 0.0)
        if with_res:
            z = z + xr_ref[...]
        z_ref[...] = z
        y = _dot(z, w_ref[...]) + b_ref[...]
        if d_pad > d_out:
            y = jnp.concatenate(
                [y, jnp.zeros((_BLK, d_pad - d_out), jnp.float32)], axis=1)
        y_ref[...] = y

    in_specs = [_rowspec(d), _rowspec(d), _rowspec(8)]
    args = [ssum[:N], ssum[N:], inv]
    if with_res:
        in_specs.append(_rowspec(d))
        args.append(x_res)
    in_specs += [pl.BlockSpec((d, d_out), lambda i: (0, 0)),
                 pl.BlockSpec((1, d_out), lambda i: (0, 0))]
    args += [W, b.reshape(1, -1)]

    return pl.pallas_call(
        body,
        grid=(N // _BLK,),
        in_specs=in_specs,
        out_specs=[_rowspec(d), _rowspec(d_pad)],
        out_shape=[jax.ShapeDtypeStruct((N, d), jnp.float32),
                   jax.ShapeDtypeStruct((N, d_pad), jnp.float32)],
    )(*args)


def kernel(X, edge_index, W1, b1, W2, b2, W3, b3):
    vids = edge_index[0].reshape(NC * NS * NHV, NH, K)
    eids = edge_index[1].reshape(NC * NS * NHV, NH, K)
    vids_d = edge_index[0].reshape(NC * NS, NCH, K)
    eids_d = edge_index[1].reshape(NC * NS, NCH, K)
    zeros_pad = jnp.zeros((N, 128), jnp.float32)
    zeros_cnt = jnp.zeros((N, 128), jnp.float32)
    ones_k = jnp.ones((K, 128), jnp.float32)

    cnt_e, cnt_v = _sc_degrees(eids_d, vids_d, ones_k, zeros_cnt)
    inv_e = _tc_inv(cnt_e)
    inv_v = _tc_inv(cnt_v)

    y1 = _tc_matmul(X, W1, b1, 128)
    s = _sc_segment_sum(y1, vids, eids, zeros_pad)
    e1 = _tc_scale(s, inv_e, relu=False)
    s = _sc_segment_sum(e1, eids, vids, zeros_pad)
    x1, y2 = _tc_boundary(s, inv_v, None, W2, b2, 128)

    s = _sc_segment_sum(y2, vids, eids, zeros_pad)
    e2 = _tc_scale(s, inv_e, relu=False)
    s = _sc_segment_sum(e2, eids, vids, zeros_pad)
    _, y3 = _tc_boundary(s, inv_v, x1, W3, b3, 128)

    s = _sc_segment_sum(y3, vids, eids, zeros_pad)
    e3 = _tc_scale(s, inv_e, relu=False)
    s = _sc_segment_sum(e3, eids, vids, zeros_pad)
    x3 = _tc_scale(s, inv_v, relu=True, d_out=64)
    return x3
